# Initial kernel scaffold; baseline (speedup 1.0000x reference)
import jax
import jax.numpy as jnp
from jax.experimental import pallas as pl


def kernel(tgt, memory, sa_in_w, sa_in_b, sa_out_w, sa_out_b, ca_in_w, ca_in_b, ca_out_w, ca_out_b, ln1_g, ln1_b, ln2_g, ln2_b, ln3_g, ln3_b, gate_w, w1, b1, w2, b2):
    raise NotImplementedError("write your pallas kernel here")


# speedup vs baseline: 1.2507x; 1.2507x over previous
import functools

import jax
import jax.numpy as jnp
from jax import lax
from jax.experimental import pallas as pl
from jax.experimental.pallas import tpu as pltpu
from jax.experimental.pallas import tpu_sc as plsc

D = 768
H = 12
DH = 64
E = 8
F = 3072
S = 2048
BM = 256
NT = (2 * S) // BM + E
P = NT * BM
BF = 768
NF = F // BF
BQ = 512

_NC = 2
_NS = 16
_NW = _NC * _NS


@functools.cache
def _sc_mesh():
    return plsc.VectorSubcoreMesh(core_axis_name="c", subcore_axis_name="s")


def _mm_nt_body(x_ref, w_ref, b_ref, o_ref):
    y = lax.dot_general(x_ref[...], w_ref[...], (((1,), (1,)), ((), ())),
                        preferred_element_type=jnp.float32)
    o_ref[...] = y + b_ref[...]


def _mm_nt(x, w, b, bm=256):
    M, K = x.shape
    N = w.shape[0]
    return pl.pallas_call(
        _mm_nt_body,
        grid=(M // bm,),
        in_specs=[
            pl.BlockSpec((bm, K), lambda i: (i, 0)),
            pl.BlockSpec((N, K), lambda i: (0, 0)),
            pl.BlockSpec((1, N), lambda i: (0, 0)),
        ],
        out_specs=pl.BlockSpec((bm, N), lambda i: (i, 0)),
        out_shape=jax.ShapeDtypeStruct((M, N), jnp.float32),
    )(x, w, b.reshape(1, N))


def _mm_gate_body(x_ref, w_ref, b_ref, gw_ref, y_ref, lg_ref):
    y = lax.dot_general(x_ref[...], w_ref[...], (((1,), (1,)), ((), ())),
                        preferred_element_type=jnp.float32)
    y = y + b_ref[...]
    y_ref[...] = y
    lg_ref[...] = jnp.dot(y, gw_ref[...], preferred_element_type=jnp.float32)


def _mm_nt_gate(x, w, b, gw, bm=256):
    M, K = x.shape
    N = w.shape[0]
    return pl.pallas_call(
        _mm_gate_body,
        grid=(M // bm,),
        in_specs=[
            pl.BlockSpec((bm, K), lambda i: (i, 0)),
            pl.BlockSpec((N, K), lambda i: (0, 0)),
            pl.BlockSpec((1, N), lambda i: (0, 0)),
            pl.BlockSpec((K, E), lambda i: (0, 0)),
        ],
        out_specs=[
            pl.BlockSpec((bm, N), lambda i: (i, 0)),
            pl.BlockSpec((bm, E), lambda i: (i, 0)),
        ],
        out_shape=[
            jax.ShapeDtypeStruct((M, N), jnp.float32),
            jax.ShapeDtypeStruct((M, E), jnp.float32),
        ],
    )(x, w, b.reshape(1, N), gw)


def _attn_body(q_ref, k_ref, v_ref, o_ref):
    q = q_ref[0]
    k = k_ref[0]
    v = v_ref[0]
    s = lax.dot_general(q, k, (((1,), (1,)), ((), ())),
                        preferred_element_type=jnp.float32) * 0.125
    m = jnp.max(s, axis=-1, keepdims=True)
    p = jnp.exp(s - m)
    p = p / jnp.sum(p, axis=-1, keepdims=True)
    o = jnp.dot(p, v, preferred_element_type=jnp.float32)
    o_ref[0] = o


def _attn(qh, kvh, qoff, koff, voff):
    sq = qh.shape[1]
    skv = kvh.shape[1]
    return pl.pallas_call(
        _attn_body,
        grid=(H, sq // BQ),
        in_specs=[
            pl.BlockSpec((1, BQ, DH), lambda h, i: (qoff + h, i, 0)),
            pl.BlockSpec((1, skv, DH), lambda h, i: (koff + h, 0, 0)),
            pl.BlockSpec((1, skv, DH), lambda h, i: (voff + h, 0, 0)),
        ],
        out_specs=pl.BlockSpec((1, BQ, DH), lambda h, i: (h, i, 0)),
        out_shape=jax.ShapeDtypeStruct((H, sq, DH), jnp.float32),
    )(qh, kvh, kvh)


def _route_body(lg_ref, pos0_ref, pos1_ref, g0_ref, g1_ref, cnt_ref,
                start_ref, aux_ref):
    lg = lg_ref[...]
    m = jnp.max(lg, axis=-1, keepdims=True)
    ex = jnp.exp(lg - m)
    probs = ex / jnp.sum(ex, axis=-1, keepdims=True)
    ecol = lax.broadcasted_iota(jnp.int32, (S, E), 1)

    p0 = jnp.max(probs, axis=-1, keepdims=True)
    i0 = jnp.min(jnp.where(probs == p0, ecol, E), axis=-1, keepdims=True)
    one0 = (ecol == i0).astype(jnp.float32)
    probs1 = jnp.where(ecol == i0, -1.0, probs)
    p1 = jnp.max(probs1, axis=-1, keepdims=True)
    i1 = jnp.min(jnp.where(probs1 == p1, ecol, E), axis=-1, keepdims=True)
    one1 = (ecol == i1).astype(jnp.float32)
    cnt = one0 + one1

    den = p0 + p1
    g0_ref[...] = p0 / den
    g1_ref[...] = p1 / den

    totals = jnp.sum(cnt, axis=0, keepdims=True)
    tiles_e = jnp.ceil(totals * (1.0 / BM))
    padc = tiles_e * BM
    er = lax.broadcasted_iota(jnp.int32, (E, E), 0)
    ec = lax.broadcasted_iota(jnp.int32, (E, E), 1)
    upper = (er < ec).astype(jnp.float32)
    starts = jnp.dot(padc, upper, preferred_element_type=jnp.float32)
    cnt_ref[...] = totals.astype(jnp.int32)
    start_ref[...] = starts.astype(jnp.int32)

    nb = S // 256
    for b in range(nb):
        rowi = lax.broadcasted_iota(jnp.int32, (256, S), 0) + b * 256
        coli = lax.broadcasted_iota(jnp.int32, (256, S), 1)
        mb = (coli < rowi).astype(jnp.float32)
        c_b = jnp.dot(mb, cnt, preferred_element_type=jnp.float32)
        sl = slice(b * 256, (b + 1) * 256)
        one0_b = one0[sl, :]
        one1_b = one1[sl, :]
        pos0_b = (jnp.sum(one0_b * (starts + c_b), axis=-1, keepdims=True))
        pos1_b = (jnp.sum(one1_b * (starts + c_b), axis=-1, keepdims=True))
        pos0_ref[sl, :] = pos0_b.astype(jnp.int32)
        pos1_ref[sl, :] = pos1_b.astype(jnp.int32)

    me = jnp.sum(probs, axis=0, keepdims=True) * (1.0 / S)
    ce = jnp.sum(one0, axis=0, keepdims=True) * (1.0 / S)
    aux_ref[...] = 0.01 * E * jnp.sum(me * ce, keepdims=True).reshape(1, 1)


def _route(logits):
    return pl.pallas_call(
        _route_body,
        grid=(1,),
        in_specs=[pl.BlockSpec((S, E), lambda i: (0, 0))],
        out_specs=[
            pl.BlockSpec((S, 1), lambda i: (0, 0)),
            pl.BlockSpec((S, 1), lambda i: (0, 0)),
            pl.BlockSpec((S, 1), lambda i: (0, 0)),
            pl.BlockSpec((S, 1), lambda i: (0, 0)),
            pl.BlockSpec((1, E), lambda i: (0, 0)),
            pl.BlockSpec((1, E), lambda i: (0, 0)),
            pl.BlockSpec((1, 1), lambda i: (0, 0)),
        ],
        out_shape=[
            jax.ShapeDtypeStruct((S, 1), jnp.int32),
            jax.ShapeDtypeStruct((S, 1), jnp.int32),
            jax.ShapeDtypeStruct((S, 1), jnp.float32),
            jax.ShapeDtypeStruct((S, 1), jnp.float32),
            jax.ShapeDtypeStruct((1, E), jnp.int32),
            jax.ShapeDtypeStruct((1, E), jnp.int32),
            jax.ShapeDtypeStruct((1, 1), jnp.float32),
        ],
    )(logits)


def _tok_body(p0_ref, p1_ref, tok_ref):
    i = pl.program_id(0)
    prow = lax.broadcasted_iota(jnp.int32, (256, S), 0) + i * 256
    m = ((p0_ref[...] == prow).astype(jnp.float32)
         + (p1_ref[...] == prow).astype(jnp.float32))
    t = lax.broadcasted_iota(jnp.int32, (S, 1), 0).astype(jnp.float32)
    tok = jnp.dot(m, t, preferred_element_type=jnp.float32)
    tok_ref[...] = tok.astype(jnp.int32)


def _build_tok(pos0, pos1):
    return pl.pallas_call(
        _tok_body,
        grid=(P // 256,),
        in_specs=[
            pl.BlockSpec((1, S), lambda i: (0, 0)),
            pl.BlockSpec((1, S), lambda i: (0, 0)),
        ],
        out_specs=pl.BlockSpec((256, 1), lambda i: (i, 0)),
        out_shape=jax.ShapeDtypeStruct((P, 1), jnp.int32),
    )(pos0.reshape(1, S), pos1.reshape(1, S))


def _sc_gather_rows(table, idx):
    n_rows = idx.shape[0]
    width = table.shape[1]
    rows_per_w = n_rows // _NW
    chunk = 96 if rows_per_w % 96 == 0 else 64
    assert rows_per_w % chunk == 0 and chunk % 8 == 0

    @functools.partial(
        pl.kernel, mesh=_sc_mesh(),
        out_type=jax.ShapeDtypeStruct((n_rows, width), jnp.float32),
        scratch_types=[pltpu.VMEM((chunk,), jnp.int32),
                       pltpu.VMEM((chunk, width), jnp.float32),
                       pltpu.SemaphoreType.DMA],
    )
    def k(table_hbm, idx_hbm, out_hbm, idx_v, rows_v, sem):
        wid = lax.axis_index("s") * _NC + lax.axis_index("c")
        base = wid * rows_per_w
        for c in range(rows_per_w // chunk):
            off = base + c * chunk
            pltpu.sync_copy(idx_hbm.at[pl.ds(off, chunk)], idx_v)
            pltpu.async_copy(table_hbm.at[idx_v], rows_v, sem).wait()
            pltpu.sync_copy(rows_v, out_hbm.at[pl.ds(off, chunk)])

    return k(table, idx)


_SQRT_HALF = 0.7071067811865476


def _ffn_body(emap_ref, x_ref, w1_ref, b1_ref, w2_ref, b2_ref, y_ref):
    f = pl.program_id(1)
    h = jnp.dot(x_ref[...], w1_ref[0], preferred_element_type=jnp.float32)
    h = h + b1_ref[0]
    h = 0.5 * h * (1.0 + lax.erf(h * _SQRT_HALF))
    part = jnp.dot(h, w2_ref[0], preferred_element_type=jnp.float32)

    @pl.when(f == 0)
    def _():
        y_ref[...] = part + b2_ref[0]

    @pl.when(f != 0)
    def _():
        y_ref[...] += part


def _ffn(emap, xs, w1, b1, w2, b2):
    grid_spec = pltpu.PrefetchScalarGridSpec(
        num_scalar_prefetch=1,
        grid=(NT, NF),
        in_specs=[
            pl.BlockSpec((BM, D), lambda t, f, emap: (t, 0)),
            pl.BlockSpec((1, D, BF), lambda t, f, emap: (emap[t], 0, f)),
            pl.BlockSpec((1, 1, BF), lambda t, f, emap: (emap[t], 0, f)),
            pl.BlockSpec((1, BF, D), lambda t, f, emap: (emap[t], f, 0)),
            pl.BlockSpec((1, 1, D), lambda t, f, emap: (emap[t], 0, 0)),
        ],
        out_specs=pl.BlockSpec((BM, D), lambda t, f, emap: (t, 0)),
    )
    return pl.pallas_call(
        _ffn_body,
        grid_spec=grid_spec,
        out_shape=jax.ShapeDtypeStruct((P, D), jnp.float32),
    )(emap, xs, w1, b1.reshape(E, 1, F), w2, b2.reshape(E, 1, D))


def _ln(x, g, b):
    m = jnp.mean(x, axis=-1, keepdims=True)
    xc = x - m
    v = jnp.mean(xc * xc, axis=-1, keepdims=True)
    return xc * lax.rsqrt(v + 1e-5) * g + b


def _comb_body(r0_ref, r1_ref, g0_ref, g1_ref, l1g, l1b, l2g, l2b, l3g, l3b,
               o_ref):
    x = g0_ref[...] * r0_ref[...] + g1_ref[...] * r1_ref[...]
    x = _ln(x, l1g[...], l1b[...])
    x = _ln(x, l2g[...], l2b[...])
    x = _ln(x, l3g[...], l3b[...])
    o_ref[...] = x


def _combine(r, g0, g1, lns, bm=256):
    ln_specs = [pl.BlockSpec((1, D), lambda i: (0, 0)) for _ in range(6)]
    return pl.pallas_call(
        _comb_body,
        grid=(S // bm,),
        in_specs=[
            pl.BlockSpec((bm, D), lambda i: (i, 0)),
            pl.BlockSpec((bm, D), lambda i: (i + S // bm, 0)),
            pl.BlockSpec((bm, 1), lambda i: (i, 0)),
            pl.BlockSpec((bm, 1), lambda i: (i, 0)),
        ] + ln_specs,
        out_specs=pl.BlockSpec((bm, D), lambda i: (i, 0)),
        out_shape=jax.ShapeDtypeStruct((S, D), jnp.float32),
    )(r, r, g0, g1, *[p.reshape(1, D) for p in lns])


def kernel(tgt, memory, sa_in_w, sa_in_b, sa_out_w, sa_out_b, ca_in_w,
           ca_in_b, ca_out_w, ca_out_b, ln1_g, ln1_b, ln2_g, ln2_b, ln3_g,
           ln3_b, gate_w, w1, b1, w2, b2):
    x0 = tgt.reshape(S, D)
    mem = memory.reshape(S, D)

    qkv = _mm_nt(x0, sa_in_w, sa_in_b)
    qkvh = qkv.reshape(S, 3 * H, DH).transpose(1, 0, 2)
    o1 = _attn(qkvh, qkvh, qoff=0, koff=H, voff=2 * H)
    x1 = _mm_nt(o1.transpose(1, 0, 2).reshape(S, D), sa_out_w, sa_out_b)

    q_ca = _mm_nt(x1, ca_in_w[:D], ca_in_b[:D])
    kv_ca = _mm_nt(mem, ca_in_w[D:], ca_in_b[D:])
    o2 = _attn(q_ca.reshape(S, H, DH).transpose(1, 0, 2),
               kv_ca.reshape(S, 2 * H, DH).transpose(1, 0, 2),
               qoff=0, koff=0, voff=H)
    x2, logits = _mm_nt_gate(o2.transpose(1, 0, 2).reshape(S, D),
                             ca_out_w, ca_out_b, gate_w)

    pos0, pos1, g0, g1, counts, starts, aux = _route(logits)
    tile_starts = starts[0] // BM
    j = jnp.arange(NT, dtype=jnp.int32)
    emap = jnp.sum((j[:, None] >= tile_starts[None, :]).astype(jnp.int32),
                   axis=1) - 1

    tok = _build_tok(pos0, pos1).reshape(P)
    xs = _sc_gather_rows(x2, tok)
    y = _ffn(emap, xs, w1, b1, w2, b2)

    pos01 = jnp.concatenate([pos0.reshape(S), pos1.reshape(S)])
    r = _sc_gather_rows(y, pos01)
    out = _combine(r, g0, g1, (ln1_g, ln1_b, ln2_g, ln2_b, ln3_g, ln3_b))

    return out.reshape(S, 1, D), aux.reshape(())


# speedup vs baseline: 1.7211x; 1.3761x over previous
import functools

import jax
import jax.numpy as jnp
from jax import lax
from jax.experimental import pallas as pl
from jax.experimental.pallas import tpu as pltpu
from jax.experimental.pallas import tpu_sc as plsc

D = 768
H = 12
DH = 64
E = 8
F = 3072
S = 2048
BM = 128
NT = (2 * S) // BM + E
P = NT * BM
BQ = 512

_NC = 2
_NS = 16
_NW = _NC * _NS


@functools.cache
def _sc_mesh():
    return plsc.VectorSubcoreMesh(core_axis_name="c", subcore_axis_name="s")


def _mm_nt_body(x_ref, w_ref, b_ref, o_ref):
    y = lax.dot_general(x_ref[...], w_ref[...], (((1,), (1,)), ((), ())),
                        preferred_element_type=jnp.float32)
    o_ref[...] = y + b_ref[...]


def _mm_nt(x, w, b, bm=256):
    M, K = x.shape
    N = w.shape[0]
    return pl.pallas_call(
        _mm_nt_body,
        grid=(M // bm,),
        in_specs=[
            pl.BlockSpec((bm, K), lambda i: (i, 0)),
            pl.BlockSpec((N, K), lambda i: (0, 0)),
            pl.BlockSpec((1, N), lambda i: (0, 0)),
        ],
        out_specs=pl.BlockSpec((bm, N), lambda i: (i, 0)),
        out_shape=jax.ShapeDtypeStruct((M, N), jnp.float32),
    )(x, w, b.reshape(1, N))


def _mm_gate_body(x_ref, w_ref, b_ref, gw_ref, y_ref, lg_ref):
    y = lax.dot_general(x_ref[...], w_ref[...], (((1,), (1,)), ((), ())),
                        preferred_element_type=jnp.float32)
    y = y + b_ref[...]
    y_ref[...] = y
    lg_ref[...] = jnp.dot(y, gw_ref[...], preferred_element_type=jnp.float32)


def _mm_nt_gate(x, w, b, gw, bm=256):
    M, K = x.shape
    N = w.shape[0]
    return pl.pallas_call(
        _mm_gate_body,
        grid=(M // bm,),
        in_specs=[
            pl.BlockSpec((bm, K), lambda i: (i, 0)),
            pl.BlockSpec((N, K), lambda i: (0, 0)),
            pl.BlockSpec((1, N), lambda i: (0, 0)),
            pl.BlockSpec((K, E), lambda i: (0, 0)),
        ],
        out_specs=[
            pl.BlockSpec((bm, N), lambda i: (i, 0)),
            pl.BlockSpec((bm, E), lambda i: (i, 0)),
        ],
        out_shape=[
            jax.ShapeDtypeStruct((M, N), jnp.float32),
            jax.ShapeDtypeStruct((M, E), jnp.float32),
        ],
    )(x, w, b.reshape(1, N), gw)


def _attn_body(q_ref, k_ref, v_ref, o_ref):
    q = q_ref[0]
    k = k_ref[0]
    v = v_ref[0]
    s = lax.dot_general(q, k, (((1,), (1,)), ((), ())),
                        preferred_element_type=jnp.float32) * 0.125
    m = jnp.max(s, axis=-1, keepdims=True)
    p = jnp.exp(s - m)
    r = 1.0 / jnp.sum(p, axis=-1, keepdims=True)
    o = jnp.dot(p, v, preferred_element_type=jnp.float32)
    o_ref[0] = o * r


def _attn(qh, kvh, qoff, koff, voff):
    sq = qh.shape[1]
    skv = kvh.shape[1]
    return pl.pallas_call(
        _attn_body,
        grid=(H, sq // BQ),
        in_specs=[
            pl.BlockSpec((1, BQ, DH), lambda h, i: (qoff + h, i, 0)),
            pl.BlockSpec((1, skv, DH), lambda h, i: (koff + h, 0, 0)),
            pl.BlockSpec((1, skv, DH), lambda h, i: (voff + h, 0, 0)),
        ],
        out_specs=pl.BlockSpec((1, BQ, DH), lambda h, i: (h, i, 0)),
        out_shape=jax.ShapeDtypeStruct((H, sq, DH), jnp.float32),
    )(qh, kvh, kvh)


def _route_body(lg_ref, pos0_ref, pos1_ref, g0_ref, g1_ref, cnt_ref,
                start_ref, aux_ref):
    lg = lg_ref[...]
    m = jnp.max(lg, axis=-1, keepdims=True)
    ex = jnp.exp(lg - m)
    probs = ex / jnp.sum(ex, axis=-1, keepdims=True)
    ecol = lax.broadcasted_iota(jnp.int32, (S, E), 1)

    p0 = jnp.max(probs, axis=-1, keepdims=True)
    i0 = jnp.min(jnp.where(probs == p0, ecol, E), axis=-1, keepdims=True)
    one0 = (ecol == i0).astype(jnp.float32)
    probs1 = jnp.where(ecol == i0, -1.0, probs)
    p1 = jnp.max(probs1, axis=-1, keepdims=True)
    i1 = jnp.min(jnp.where(probs1 == p1, ecol, E), axis=-1, keepdims=True)
    one1 = (ecol == i1).astype(jnp.float32)
    cnt = one0 + one1

    den = p0 + p1
    g0_ref[...] = p0 / den
    g1_ref[...] = p1 / den

    totals = jnp.sum(cnt, axis=0, keepdims=True)
    tiles_e = jnp.ceil(totals * (1.0 / BM))
    padc = tiles_e * BM
    er = lax.broadcasted_iota(jnp.int32, (E, E), 0)
    ec = lax.broadcasted_iota(jnp.int32, (E, E), 1)
    upper = (er < ec).astype(jnp.float32)
    starts = jnp.dot(padc, upper, preferred_element_type=jnp.float32)
    cnt_ref[...] = totals.astype(jnp.int32)
    start_ref[...] = starts.astype(jnp.int32)

    nb = S // 256
    for b in range(nb):
        rowi = lax.broadcasted_iota(jnp.int32, (256, S), 0) + b * 256
        coli = lax.broadcasted_iota(jnp.int32, (256, S), 1)
        mb = (coli < rowi).astype(jnp.float32)
        c_b = jnp.dot(mb, cnt, preferred_element_type=jnp.float32)
        sl = slice(b * 256, (b + 1) * 256)
        one0_b = one0[sl, :]
        one1_b = one1[sl, :]
        pos0_b = (jnp.sum(one0_b * (starts + c_b), axis=-1, keepdims=True))
        pos1_b = (jnp.sum(one1_b * (starts + c_b), axis=-1, keepdims=True))
        pos0_ref[sl, :] = pos0_b.astype(jnp.int32)
        pos1_ref[sl, :] = pos1_b.astype(jnp.int32)

    me = jnp.sum(probs, axis=0, keepdims=True) * (1.0 / S)
    ce = jnp.sum(one0, axis=0, keepdims=True) * (1.0 / S)
    aux_ref[...] = 0.01 * E * jnp.sum(me * ce, keepdims=True).reshape(1, 1)


def _route(logits):
    return pl.pallas_call(
        _route_body,
        grid=(1,),
        in_specs=[pl.BlockSpec((S, E), lambda i: (0, 0))],
        out_specs=[
            pl.BlockSpec((S, 1), lambda i: (0, 0)),
            pl.BlockSpec((S, 1), lambda i: (0, 0)),
            pl.BlockSpec((S, 1), lambda i: (0, 0)),
            pl.BlockSpec((S, 1), lambda i: (0, 0)),
            pl.BlockSpec((1, E), lambda i: (0, 0)),
            pl.BlockSpec((1, E), lambda i: (0, 0)),
            pl.BlockSpec((1, 1), lambda i: (0, 0)),
        ],
        out_shape=[
            jax.ShapeDtypeStruct((S, 1), jnp.int32),
            jax.ShapeDtypeStruct((S, 1), jnp.int32),
            jax.ShapeDtypeStruct((S, 1), jnp.float32),
            jax.ShapeDtypeStruct((S, 1), jnp.float32),
            jax.ShapeDtypeStruct((1, E), jnp.int32),
            jax.ShapeDtypeStruct((1, E), jnp.int32),
            jax.ShapeDtypeStruct((1, 1), jnp.float32),
        ],
    )(logits)


def _sc_dispatch(x, pos0, pos1):
    rows_per_w = S // _NW

    @functools.partial(
        pl.kernel, mesh=_sc_mesh(),
        out_type=jax.ShapeDtypeStruct((P, D), jnp.float32),
        scratch_types=[pltpu.VMEM((rows_per_w,), jnp.int32),
                       pltpu.VMEM((rows_per_w,), jnp.int32),
                       pltpu.VMEM((rows_per_w, D), jnp.float32),
                       pltpu.SemaphoreType.DMA,
                       pltpu.SemaphoreType.DMA],
    )
    def k(x_hbm, p0_hbm, p1_hbm, out_hbm, i0_v, i1_v, rows_v, s0, s1):
        wid = lax.axis_index("s") * _NC + lax.axis_index("c")
        base = wid * rows_per_w
        pltpu.sync_copy(p0_hbm.at[pl.ds(base, rows_per_w)], i0_v)
        pltpu.sync_copy(p1_hbm.at[pl.ds(base, rows_per_w)], i1_v)
        pltpu.sync_copy(x_hbm.at[pl.ds(base, rows_per_w)], rows_v)
        c0 = pltpu.async_copy(rows_v, out_hbm.at[i0_v], s0)
        c1 = pltpu.async_copy(rows_v, out_hbm.at[i1_v], s1)
        c0.wait()
        c1.wait()

    return k(x, pos0.reshape(S), pos1.reshape(S))


def _sc_gather_rows(table, idx):
    n_rows = idx.shape[0]
    width = table.shape[1]
    rows_per_w = n_rows // _NW
    half = rows_per_w // 2
    assert half % 8 == 0 and half <= 128

    @functools.partial(
        pl.kernel, mesh=_sc_mesh(),
        out_type=jax.ShapeDtypeStruct((n_rows, width), jnp.float32),
        scratch_types=[pltpu.VMEM((rows_per_w,), jnp.int32),
                       pltpu.VMEM((half, width), jnp.float32),
                       pltpu.VMEM((half, width), jnp.float32),
                       pltpu.SemaphoreType.DMA,
                       pltpu.SemaphoreType.DMA,
                       pltpu.SemaphoreType.DMA,
                       pltpu.SemaphoreType.DMA],
    )
    def k(table_hbm, idx_hbm, out_hbm, idx_v, b0, b1, g0, g1, s0, s1):
        wid = lax.axis_index("s") * _NC + lax.axis_index("c")
        base = wid * rows_per_w
        pltpu.sync_copy(idx_hbm.at[pl.ds(base, rows_per_w)], idx_v)
        ga0 = pltpu.async_copy(table_hbm.at[idx_v.at[pl.ds(0, half)]], b0, g0)
        ga1 = pltpu.async_copy(table_hbm.at[idx_v.at[pl.ds(half, half)]],
                               b1, g1)
        ga0.wait()
        st0 = pltpu.async_copy(b0, out_hbm.at[pl.ds(base, half)], s0)
        ga1.wait()
        st1 = pltpu.async_copy(b1, out_hbm.at[pl.ds(base + half, half)], s1)
        st0.wait()
        st1.wait()

    return k(table, idx)


_SQRT_HALF = 0.7071067811865476


def _ffn_body(emap_ref, x_ref, w1_ref, b1_ref, w2_ref, b2_ref, y_ref):
    h = jnp.dot(x_ref[...], w1_ref[0], preferred_element_type=jnp.float32)
    h = h + b1_ref[0]
    h = 0.5 * h * (1.0 + lax.erf(h * _SQRT_HALF))
    y = jnp.dot(h, w2_ref[0], preferred_element_type=jnp.float32)
    y_ref[...] = y + b2_ref[0]


def _ffn(emap, xs, w1, b1, w2, b2):
    grid_spec = pltpu.PrefetchScalarGridSpec(
        num_scalar_prefetch=1,
        grid=(NT,),
        in_specs=[
            pl.BlockSpec((BM, D), lambda t, emap: (t, 0)),
            pl.BlockSpec((1, D, F), lambda t, emap: (emap[t], 0, 0)),
            pl.BlockSpec((1, 1, F), lambda t, emap: (emap[t], 0, 0)),
            pl.BlockSpec((1, F, D), lambda t, emap: (emap[t], 0, 0)),
            pl.BlockSpec((1, 1, D), lambda t, emap: (emap[t], 0, 0)),
        ],
        out_specs=pl.BlockSpec((BM, D), lambda t, emap: (t, 0)),
    )
    return pl.pallas_call(
        _ffn_body,
        grid_spec=grid_spec,
        out_shape=jax.ShapeDtypeStruct((P, D), jnp.float32),
    )(emap, xs, w1, b1.reshape(E, 1, F), w2, b2.reshape(E, 1, D))


def _ln(x, g, b):
    m = jnp.mean(x, axis=-1, keepdims=True)
    xc = x - m
    v = jnp.mean(xc * xc, axis=-1, keepdims=True)
    return xc * lax.rsqrt(v + 1e-5) * g + b


def _comb_body(r0_ref, r1_ref, g0_ref, g1_ref, l1g, l1b, l2g, l2b, l3g, l3b,
               o_ref):
    x = g0_ref[...] * r0_ref[...] + g1_ref[...] * r1_ref[...]
    x = _ln(x, l1g[...], l1b[...])
    x = _ln(x, l2g[...], l2b[...])
    x = _ln(x, l3g[...], l3b[...])
    o_ref[...] = x


def _combine(r, g0, g1, lns, bm=256):
    ln_specs = [pl.BlockSpec((1, D), lambda i: (0, 0)) for _ in range(6)]
    return pl.pallas_call(
        _comb_body,
        grid=(S // bm,),
        in_specs=[
            pl.BlockSpec((bm, D), lambda i: (i, 0)),
            pl.BlockSpec((bm, D), lambda i: (i + S // bm, 0)),
            pl.BlockSpec((bm, 1), lambda i: (i, 0)),
            pl.BlockSpec((bm, 1), lambda i: (i, 0)),
        ] + ln_specs,
        out_specs=pl.BlockSpec((bm, D), lambda i: (i, 0)),
        out_shape=jax.ShapeDtypeStruct((S, D), jnp.float32),
    )(r, r, g0, g1, *[p.reshape(1, D) for p in lns])


def kernel(tgt, memory, sa_in_w, sa_in_b, sa_out_w, sa_out_b, ca_in_w,
           ca_in_b, ca_out_w, ca_out_b, ln1_g, ln1_b, ln2_g, ln2_b, ln3_g,
           ln3_b, gate_w, w1, b1, w2, b2):
    x0 = tgt.reshape(S, D)
    mem = memory.reshape(S, D)

    qkv = _mm_nt(x0, sa_in_w, sa_in_b)
    qkvh = qkv.reshape(S, 3 * H, DH).transpose(1, 0, 2)
    o1 = _attn(qkvh, qkvh, qoff=0, koff=H, voff=2 * H)
    x1 = _mm_nt(o1.transpose(1, 0, 2).reshape(S, D), sa_out_w, sa_out_b)

    q_ca = _mm_nt(x1, ca_in_w[:D], ca_in_b[:D])
    kv_ca = _mm_nt(mem, ca_in_w[D:], ca_in_b[D:])
    o2 = _attn(q_ca.reshape(S, H, DH).transpose(1, 0, 2),
               kv_ca.reshape(S, 2 * H, DH).transpose(1, 0, 2),
               qoff=0, koff=0, voff=H)
    x2, logits = _mm_nt_gate(o2.transpose(1, 0, 2).reshape(S, D),
                             ca_out_w, ca_out_b, gate_w)

    pos0, pos1, g0, g1, counts, starts, aux = _route(logits)
    tile_starts = starts[0] // BM
    j = jnp.arange(NT, dtype=jnp.int32)
    emap = jnp.sum((j[:, None] >= tile_starts[None, :]).astype(jnp.int32),
                   axis=1) - 1

    xs = _sc_dispatch(x2, pos0, pos1)
    y = _ffn(emap, xs, w1, b1, w2, b2)

    pos01 = jnp.concatenate([pos0.reshape(S), pos1.reshape(S)])
    r = _sc_gather_rows(y, pos01)
    out = _combine(r, g0, g1, (ln1_g, ln1_b, ln2_g, ln2_b, ln3_g, ln3_b))

    return out.reshape(S, 1, D), aux.reshape(())


# speedup vs baseline: 2.3621x; 1.3725x over previous
import functools

import jax
import jax.numpy as jnp
from jax import lax
from jax.experimental import pallas as pl
from jax.experimental.pallas import tpu as pltpu
from jax.experimental.pallas import tpu_sc as plsc

D = 768
H = 12
DH = 64
E = 8
F = 3072
S = 2048
BM = 128
NT = (2 * S) // BM + E
P = NT * BM
BQ = 512

_NC = 2
_NS = 16
_NW = _NC * _NS


@functools.cache
def _sc_mesh():
    return plsc.VectorSubcoreMesh(core_axis_name="c", subcore_axis_name="s")


def _mm_nt_body(x_ref, w_ref, b_ref, o_ref):
    y = lax.dot_general(x_ref[...], w_ref[...], (((1,), (1,)), ((), ())),
                        preferred_element_type=jnp.float32)
    o_ref[...] = y + b_ref[...]


def _mm_nt(x, w, b, bm=256):
    M, K = x.shape
    N = w.shape[0]
    return pl.pallas_call(
        _mm_nt_body,
        grid=(M // bm,),
        in_specs=[
            pl.BlockSpec((bm, K), lambda i: (i, 0)),
            pl.BlockSpec((N, K), lambda i: (0, 0)),
            pl.BlockSpec((1, N), lambda i: (0, 0)),
        ],
        out_specs=pl.BlockSpec((bm, N), lambda i: (i, 0)),
        out_shape=jax.ShapeDtypeStruct((M, N), jnp.float32),
    )(x, w, b.reshape(1, N))


def _mm_gate_body(x_ref, w_ref, b_ref, gw_ref, y_ref, lg_ref):
    y = lax.dot_general(x_ref[...], w_ref[...], (((1,), (1,)), ((), ())),
                        preferred_element_type=jnp.float32)
    y = y + b_ref[...]
    y_ref[...] = y
    lg_ref[...] = jnp.dot(y, gw_ref[...], preferred_element_type=jnp.float32)


def _mm_nt_gate(x, w, b, gw, bm=256):
    M, K = x.shape
    N = w.shape[0]
    return pl.pallas_call(
        _mm_gate_body,
        grid=(M // bm,),
        in_specs=[
            pl.BlockSpec((bm, K), lambda i: (i, 0)),
            pl.BlockSpec((N, K), lambda i: (0, 0)),
            pl.BlockSpec((1, N), lambda i: (0, 0)),
            pl.BlockSpec((K, E), lambda i: (0, 0)),
        ],
        out_specs=[
            pl.BlockSpec((bm, N), lambda i: (i, 0)),
            pl.BlockSpec((bm, E), lambda i: (i, 0)),
        ],
        out_shape=[
            jax.ShapeDtypeStruct((M, N), jnp.float32),
            jax.ShapeDtypeStruct((M, E), jnp.float32),
        ],
    )(x, w, b.reshape(1, N), gw)


def _attn_body(q_ref, k_ref, v_ref, o_ref):
    q = q_ref[...]
    k = k_ref[...]
    v = v_ref[...]
    outs = []
    for a in range(2):
        sl = slice(a * DH, (a + 1) * DH)
        s = lax.dot_general(q[:, sl], k[:, sl], (((1,), (1,)), ((), ())),
                            preferred_element_type=jnp.float32) * 0.125
        m = jnp.max(s, axis=-1, keepdims=True)
        p = jnp.exp(s - m)
        r = 1.0 / jnp.sum(p, axis=-1, keepdims=True)
        o = jnp.dot(p, v[:, sl], preferred_element_type=jnp.float32)
        outs.append(o * r)
    o_ref[...] = jnp.concatenate(outs, axis=1)


def _attn(qm, kvm, qoff, koff, voff):
    sq = qm.shape[0]
    skv = kvm.shape[0]
    return pl.pallas_call(
        _attn_body,
        grid=(H // 2, sq // BQ),
        in_specs=[
            pl.BlockSpec((BQ, 2 * DH), lambda h, i: (i, qoff + h)),
            pl.BlockSpec((skv, 2 * DH), lambda h, i: (0, koff + h)),
            pl.BlockSpec((skv, 2 * DH), lambda h, i: (0, voff + h)),
        ],
        out_specs=pl.BlockSpec((BQ, 2 * DH), lambda h, i: (i, h)),
        out_shape=jax.ShapeDtypeStruct((sq, D), jnp.float32),
    )(qm, kvm, kvm)


def _route_body(lg_ref, pos0_ref, pos1_ref, g0_ref, g1_ref, cnt_ref,
                start_ref, aux_ref):
    lg = lg_ref[...]
    m = jnp.max(lg, axis=-1, keepdims=True)
    ex = jnp.exp(lg - m)
    probs = ex / jnp.sum(ex, axis=-1, keepdims=True)
    ecol = lax.broadcasted_iota(jnp.int32, (S, E), 1)

    p0 = jnp.max(probs, axis=-1, keepdims=True)
    i0 = jnp.min(jnp.where(probs == p0, ecol, E), axis=-1, keepdims=True)
    one0 = (ecol == i0).astype(jnp.float32)
    probs1 = jnp.where(ecol == i0, -1.0, probs)
    p1 = jnp.max(probs1, axis=-1, keepdims=True)
    i1 = jnp.min(jnp.where(probs1 == p1, ecol, E), axis=-1, keepdims=True)
    one1 = (ecol == i1).astype(jnp.float32)
    cnt = one0 + one1

    den = p0 + p1
    g0_ref[...] = p0 / den
    g1_ref[...] = p1 / den

    totals = jnp.sum(cnt, axis=0, keepdims=True)
    tiles_e = jnp.ceil(totals * (1.0 / BM))
    padc = tiles_e * BM
    er = lax.broadcasted_iota(jnp.int32, (E, E), 0)
    ec = lax.broadcasted_iota(jnp.int32, (E, E), 1)
    upper = (er < ec).astype(jnp.float32)
    starts = jnp.dot(padc, upper, preferred_element_type=jnp.float32)
    cnt_ref[...] = totals.astype(jnp.int32)
    start_ref[...] = starts.astype(jnp.int32)

    nb = S // 256
    for b in range(nb):
        rowi = lax.broadcasted_iota(jnp.int32, (256, S), 0) + b * 256
        coli = lax.broadcasted_iota(jnp.int32, (256, S), 1)
        mb = (coli < rowi).astype(jnp.float32)
        c_b = jnp.dot(mb, cnt, preferred_element_type=jnp.float32)
        sl = slice(b * 256, (b + 1) * 256)
        one0_b = one0[sl, :]
        one1_b = one1[sl, :]
        pos0_b = (jnp.sum(one0_b * (starts + c_b), axis=-1, keepdims=True))
        pos1_b = (jnp.sum(one1_b * (starts + c_b), axis=-1, keepdims=True))
        pos0_ref[sl, :] = pos0_b.astype(jnp.int32)
        pos1_ref[sl, :] = pos1_b.astype(jnp.int32)

    me = jnp.sum(probs, axis=0, keepdims=True) * (1.0 / S)
    ce = jnp.sum(one0, axis=0, keepdims=True) * (1.0 / S)
    aux_ref[...] = 0.01 * E * jnp.sum(me * ce, keepdims=True).reshape(1, 1)


def _route(logits):
    return pl.pallas_call(
        _route_body,
        grid=(1,),
        in_specs=[pl.BlockSpec((S, E), lambda i: (0, 0))],
        out_specs=[
            pl.BlockSpec((S, 1), lambda i: (0, 0)),
            pl.BlockSpec((S, 1), lambda i: (0, 0)),
            pl.BlockSpec((S, 1), lambda i: (0, 0)),
            pl.BlockSpec((S, 1), lambda i: (0, 0)),
            pl.BlockSpec((1, E), lambda i: (0, 0)),
            pl.BlockSpec((1, E), lambda i: (0, 0)),
            pl.BlockSpec((1, 1), lambda i: (0, 0)),
        ],
        out_shape=[
            jax.ShapeDtypeStruct((S, 1), jnp.int32),
            jax.ShapeDtypeStruct((S, 1), jnp.int32),
            jax.ShapeDtypeStruct((S, 1), jnp.float32),
            jax.ShapeDtypeStruct((S, 1), jnp.float32),
            jax.ShapeDtypeStruct((1, E), jnp.int32),
            jax.ShapeDtypeStruct((1, E), jnp.int32),
            jax.ShapeDtypeStruct((1, 1), jnp.float32),
        ],
    )(logits)


def _sc_dispatch(x, pos0, pos1):
    rows_per_w = S // _NW

    @functools.partial(
        pl.kernel, mesh=_sc_mesh(),
        out_type=jax.ShapeDtypeStruct((P, D), jnp.float32),
        scratch_types=[pltpu.VMEM((rows_per_w,), jnp.int32),
                       pltpu.VMEM((rows_per_w,), jnp.int32),
                       pltpu.VMEM((rows_per_w, D), jnp.float32),
                       pltpu.SemaphoreType.DMA,
                       pltpu.SemaphoreType.DMA],
    )
    def k(x_hbm, p0_hbm, p1_hbm, out_hbm, i0_v, i1_v, rows_v, s0, s1):
        wid = lax.axis_index("s") * _NC + lax.axis_index("c")
        base = wid * rows_per_w
        pltpu.sync_copy(p0_hbm.at[pl.ds(base, rows_per_w)], i0_v)
        pltpu.sync_copy(p1_hbm.at[pl.ds(base, rows_per_w)], i1_v)
        pltpu.sync_copy(x_hbm.at[pl.ds(base, rows_per_w)], rows_v)
        c0 = pltpu.async_copy(rows_v, out_hbm.at[i0_v], s0)
        c1 = pltpu.async_copy(rows_v, out_hbm.at[i1_v], s1)
        c0.wait()
        c1.wait()

    return k(x, pos0.reshape(S), pos1.reshape(S))


def _sc_gather_rows(table, idx):
    n_rows = idx.shape[0]
    width = table.shape[1]
    rows_per_w = n_rows // _NW
    half = rows_per_w // 2
    assert half % 8 == 0 and half <= 128

    @functools.partial(
        pl.kernel, mesh=_sc_mesh(),
        out_type=jax.ShapeDtypeStruct((n_rows, width), jnp.float32),
        scratch_types=[pltpu.VMEM((rows_per_w,), jnp.int32),
                       pltpu.VMEM((half, width), jnp.float32),
                       pltpu.VMEM((half, width), jnp.float32),
                       pltpu.SemaphoreType.DMA,
                       pltpu.SemaphoreType.DMA,
                       pltpu.SemaphoreType.DMA,
                       pltpu.SemaphoreType.DMA],
    )
    def k(table_hbm, idx_hbm, out_hbm, idx_v, b0, b1, g0, g1, s0, s1):
        wid = lax.axis_index("s") * _NC + lax.axis_index("c")
        base = wid * rows_per_w
        pltpu.sync_copy(idx_hbm.at[pl.ds(base, rows_per_w)], idx_v)
        ga0 = pltpu.async_copy(table_hbm.at[idx_v.at[pl.ds(0, half)]], b0, g0)
        ga1 = pltpu.async_copy(table_hbm.at[idx_v.at[pl.ds(half, half)]],
                               b1, g1)
        ga0.wait()
        st0 = pltpu.async_copy(b0, out_hbm.at[pl.ds(base, half)], s0)
        ga1.wait()
        st1 = pltpu.async_copy(b1, out_hbm.at[pl.ds(base + half, half)], s1)
        st0.wait()
        st1.wait()

    return k(table, idx)


_SQRT_HALF = 0.7071067811865476


def _ffn_body(emap_ref, x_ref, w1_ref, b1_ref, w2_ref, b2_ref, y_ref):
    h = jnp.dot(x_ref[...], w1_ref[0], preferred_element_type=jnp.float32)
    h = h + b1_ref[0]
    h = 0.5 * h * (1.0 + lax.erf(h * _SQRT_HALF))
    y = jnp.dot(h, w2_ref[0], preferred_element_type=jnp.float32)
    y_ref[...] = y + b2_ref[0]


def _ffn(emap, xs, w1, b1, w2, b2):
    grid_spec = pltpu.PrefetchScalarGridSpec(
        num_scalar_prefetch=1,
        grid=(NT,),
        in_specs=[
            pl.BlockSpec((BM, D), lambda t, emap: (t, 0)),
            pl.BlockSpec((1, D, F), lambda t, emap: (emap[t], 0, 0)),
            pl.BlockSpec((1, 1, F), lambda t, emap: (emap[t], 0, 0)),
            pl.BlockSpec((1, F, D), lambda t, emap: (emap[t], 0, 0)),
            pl.BlockSpec((1, 1, D), lambda t, emap: (emap[t], 0, 0)),
        ],
        out_specs=pl.BlockSpec((BM, D), lambda t, emap: (t, 0)),
    )
    return pl.pallas_call(
        _ffn_body,
        grid_spec=grid_spec,
        out_shape=jax.ShapeDtypeStruct((P, D), jnp.float32),
    )(emap, xs, w1, b1.reshape(E, 1, F), w2, b2.reshape(E, 1, D))


def _ln(x, g, b):
    m = jnp.mean(x, axis=-1, keepdims=True)
    xc = x - m
    v = jnp.mean(xc * xc, axis=-1, keepdims=True)
    return xc * lax.rsqrt(v + 1e-5) * g + b


def _comb_body(r0_ref, r1_ref, g0_ref, g1_ref, l1g, l1b, l2g, l2b, l3g, l3b,
               o_ref):
    x = g0_ref[...] * r0_ref[...] + g1_ref[...] * r1_ref[...]
    x = _ln(x, l1g[...], l1b[...])
    x = _ln(x, l2g[...], l2b[...])
    x = _ln(x, l3g[...], l3b[...])
    o_ref[...] = x


def _combine(r, g0, g1, lns, bm=256):
    ln_specs = [pl.BlockSpec((1, D), lambda i: (0, 0)) for _ in range(6)]
    return pl.pallas_call(
        _comb_body,
        grid=(S // bm,),
        in_specs=[
            pl.BlockSpec((bm, D), lambda i: (i, 0)),
            pl.BlockSpec((bm, D), lambda i: (i + S // bm, 0)),
            pl.BlockSpec((bm, 1), lambda i: (i, 0)),
            pl.BlockSpec((bm, 1), lambda i: (i, 0)),
        ] + ln_specs,
        out_specs=pl.BlockSpec((bm, D), lambda i: (i, 0)),
        out_shape=jax.ShapeDtypeStruct((S, D), jnp.float32),
    )(r, r, g0, g1, *[p.reshape(1, D) for p in lns])


def kernel(tgt, memory, sa_in_w, sa_in_b, sa_out_w, sa_out_b, ca_in_w,
           ca_in_b, ca_out_w, ca_out_b, ln1_g, ln1_b, ln2_g, ln2_b, ln3_g,
           ln3_b, gate_w, w1, b1, w2, b2):
    x0 = tgt.reshape(S, D)
    mem = memory.reshape(S, D)

    qkv = _mm_nt(x0, sa_in_w, sa_in_b)
    o1 = _attn(qkv, qkv, qoff=0, koff=H // 2, voff=H)
    x1 = _mm_nt(o1, sa_out_w, sa_out_b)

    q_ca = _mm_nt(x1, ca_in_w[:D], ca_in_b[:D])
    kv_ca = _mm_nt(mem, ca_in_w[D:], ca_in_b[D:])
    o2 = _attn(q_ca, kv_ca, qoff=0, koff=0, voff=H // 2)
    x2, logits = _mm_nt_gate(o2, ca_out_w, ca_out_b, gate_w)

    pos0, pos1, g0, g1, counts, starts, aux = _route(logits)
    tile_starts = starts[0] // BM
    j = jnp.arange(NT, dtype=jnp.int32)
    emap = jnp.sum((j[:, None] >= tile_starts[None, :]).astype(jnp.int32),
                   axis=1) - 1

    xs = _sc_dispatch(x2, pos0, pos1)
    y = _ffn(emap, xs, w1, b1, w2, b2)

    pos01 = jnp.concatenate([pos0.reshape(S), pos1.reshape(S)])
    r = _sc_gather_rows(y, pos01)
    out = _combine(r, g0, g1, (ln1_g, ln1_b, ln2_g, ln2_b, ln3_g, ln3_b))

    return out.reshape(S, 1, D), aux.reshape(())
